# baseline (speedup 1.0000x reference)
import functools

import jax
import jax.numpy as jnp
from jax import lax
from jax.experimental import pallas as pl
from jax.experimental.pallas import tpu as pltpu
from jax.experimental.pallas import tpu_sc as plsc

_B, _L, _V, _D = 4, 4096, 1000, 2048
_N = _B * _L
_NC, _NS = 2, 16
_NW = _NC * _NS
_PER_W = _N // _NW
_C = 16
_G = _PER_W // _C
_LANES = 16
_DP = _D // 2


def _make_gather_sum():
    mesh = plsc.VectorSubcoreMesh(
        core_axis_name="c", subcore_axis_name="s",
        num_cores=_NC, num_subcores=_NS)

    @functools.partial(
        pl.kernel,
        out_type=jax.ShapeDtypeStruct((_N, _DP), jnp.float32),
        mesh=mesh,
        compiler_params=pltpu.CompilerParams(needs_layout_passes=False),
        scratch_types=[
            pltpu.VMEM((_PER_W,), jnp.int32),
            pltpu.VMEM((_PER_W,), jnp.int32),
            pltpu.VMEM((_C, _DP), jnp.float32),
            pltpu.VMEM((_C, _DP), jnp.float32),
            pltpu.VMEM((_C, _DP), jnp.float32),
            pltpu.VMEM((_C, _DP), jnp.float32),
            pltpu.VMEM((_C, _DP), jnp.float32),
            pltpu.VMEM((_C, _DP), jnp.float32),
            pltpu.SemaphoreType.DMA,
            pltpu.SemaphoreType.DMA,
            pltpu.SemaphoreType.DMA,
            pltpu.SemaphoreType.DMA,
            pltpu.SemaphoreType.DMA,
            pltpu.SemaphoreType.DMA,
        ],
    )
    def gather_sum(pidx_hbm, sidx_hbm, ptab_hbm, stab_hbm, out_hbm,
                   idxp_v, idxs_v, bufp0, bufp1, bufs0, bufs1, bufo0, bufo1,
                   semp0, semp1, sems0, sems1, semw0, semw1):
        bufp = (bufp0, bufp1)
        bufs = (bufs0, bufs1)
        bufo = (bufo0, bufo1)
        semp = (semp0, semp1)
        sems = (sems0, sems1)
        semw = (semw0, semw1)

        wid = lax.axis_index("s") * _NC + lax.axis_index("c")
        wbase = wid * _PER_W
        pltpu.sync_copy(pidx_hbm.at[pl.ds(wbase, _PER_W)], idxp_v)
        pltpu.sync_copy(sidx_hbm.at[pl.ds(wbase, _PER_W)], idxs_v)

        def gather_pair(g, b):
            pltpu.async_copy(
                ptab_hbm.at[idxp_v.at[pl.ds(g * _C, _C)]], bufp[b], semp[b])
            pltpu.async_copy(
                stab_hbm.at[idxs_v.at[pl.ds(g * _C, _C)]], bufs[b], sems[b])

        def wait_gather_pair(g, b):
            pltpu.make_async_copy(
                ptab_hbm.at[idxp_v.at[pl.ds(g * _C, _C)]], bufp[b], semp[b]).wait()
            pltpu.make_async_copy(
                stab_hbm.at[idxs_v.at[pl.ds(g * _C, _C)]], bufs[b], sems[b]).wait()

        gather_pair(0, 0)
        gather_pair(1, 1)

        def add_chunk(b):
            himask = jnp.uint32(0xFFFF0000).astype(jnp.int32)
            rnd = jnp.int32(0x8000)

            def row(c, carry):
                for j in range(_DP // _LANES):
                    o = j * _LANES
                    wp = plsc.bitcast(bufp[b][c, pl.ds(o, _LANES)], jnp.int32)
                    ws = plsc.bitcast(bufs[b][c, pl.ds(o, _LANES)], jnp.int32)
                    ap = plsc.bitcast(lax.shift_left(wp, 16), jnp.float32)
                    as_ = plsc.bitcast(lax.shift_left(ws, 16), jnp.float32)
                    bp = plsc.bitcast(lax.bitwise_and(wp, himask), jnp.float32)
                    bs = plsc.bitcast(lax.bitwise_and(ws, himask), jnp.float32)
                    abits = plsc.bitcast(ap + as_, jnp.int32)
                    bbits = plsc.bitcast(bp + bs, jnp.int32)
                    lo = lax.shift_right_logical(abits + rnd, 16)
                    hi = lax.bitwise_and(bbits + rnd, himask)
                    bufo[b][c, pl.ds(o, _LANES)] = plsc.bitcast(
                        lax.bitwise_or(lo, hi), jnp.float32)
                return carry
            lax.fori_loop(0, _C, row, 0)

        def outer(g2, carry):
            for b in (0, 1):
                g = g2 * 2 + b
                base = wbase + g * _C
                wait_gather_pair(g, b)
                @pl.when(g >= 2)
                def _():
                    pltpu.make_async_copy(
                        bufo[b], out_hbm.at[pl.ds(base, _C)], semw[b]).wait()
                add_chunk(b)
                pltpu.async_copy(bufo[b], out_hbm.at[pl.ds(base, _C)], semw[b])
                @pl.when(g + 2 < _G)
                def _():
                    gather_pair(g + 2, b)
            return carry

        lax.fori_loop(0, _G // 2, outer, 0)

        for b in (0, 1):
            pltpu.make_async_copy(
                bufo[b], out_hbm.at[pl.ds(wbase, _C)], semw[b]).wait()

    return gather_sum


_gather_sum = _make_gather_sum()


def _ln_body(e_ref, x_ref, y_ref, w0_ref, wy_ref, b_ref, g_ref, bt_ref, o_ref):
    e = e_ref[...].astype(jnp.float32)
    cord = x_ref[...] * w0_ref[...] + y_ref[...] * wy_ref[...] + b_ref[...]
    e = e + cord
    mean = jnp.mean(e, axis=1, keepdims=True)
    d = e - mean
    var = jnp.mean(d * d, axis=1, keepdims=True)
    o_ref[...] = d * lax.rsqrt(var + 1e-5) * g_ref[...] + bt_ref[...]


_T = 512


def _ln_call(esum, x2, y2, w0, wy, b2, g2, bt2):
    vec = pl.BlockSpec((1, _D), lambda i: (0, 0))
    return pl.pallas_call(
        _ln_body,
        grid=(_N // _T,),
        in_specs=[
            pl.BlockSpec((_T, _D), lambda i: (i, 0)),
            pl.BlockSpec((_T, 1), lambda i: (i, 0)),
            pl.BlockSpec((_T, 1), lambda i: (i, 0)),
            vec, vec, vec, vec, vec,
        ],
        out_specs=pl.BlockSpec((_T, _D), lambda i: (i, 0)),
        out_shape=jax.ShapeDtypeStruct((_N, _D), jnp.float32),
    )(esum, x2, y2, w0, wy, b2, g2, bt2)


def kernel(primary, ss, x, y, primary_table, ss_table, W_cord, b_cord, gamma, beta):
    pidx = primary.reshape(_N).astype(jnp.int32)
    sidx = ss.reshape(_N).astype(jnp.int32)
    ptab = lax.bitcast_convert_type(
        primary_table.astype(jnp.bfloat16).reshape(_V, _DP, 2), jnp.float32)
    stab = lax.bitcast_convert_type(
        ss_table.astype(jnp.bfloat16).reshape(_V, _DP, 2), jnp.float32)
    esum_pk = _gather_sum(pidx, sidx, ptab, stab)
    esum = lax.bitcast_convert_type(esum_pk, jnp.bfloat16).reshape(_N, _D)
    w0 = W_cord[0:1]
    wy = (W_cord[1] + W_cord[2])[None]
    out = _ln_call(esum, x.reshape(_N, 1), y.reshape(_N, 1),
                   w0, wy, b_cord[None], gamma[None], beta[None])
    return out.reshape(_B, _L, _D)


# speedup vs baseline: 2.8069x; 2.8069x over previous
import functools

import jax
import jax.numpy as jnp
from jax import lax
from jax.experimental import pallas as pl
from jax.experimental.pallas import tpu as pltpu
from jax.experimental.pallas import tpu_sc as plsc

_B, _L, _V, _D = 4, 4096, 1000, 2048
_N = _B * _L
_NC, _NS = 2, 16
_NW = _NC * _NS
_PER_W = _N // _NW
_C = 16
_G = _PER_W // _C
_LANES = 16
_DP = _D // 2


def _make_gather_sum():
    mesh = plsc.VectorSubcoreMesh(
        core_axis_name="c", subcore_axis_name="s",
        num_cores=_NC, num_subcores=_NS)

    @functools.partial(
        pl.kernel,
        out_type=jax.ShapeDtypeStruct((_N, _DP), jnp.float32),
        mesh=mesh,
        compiler_params=pltpu.CompilerParams(needs_layout_passes=False),
        scratch_types=[
            pltpu.VMEM((_PER_W,), jnp.int32),
            pltpu.VMEM((_PER_W,), jnp.int32),
            pltpu.VMEM((_C, _DP), jnp.float32),
            pltpu.VMEM((_C, _DP), jnp.float32),
            pltpu.VMEM((_C, _DP), jnp.float32),
            pltpu.VMEM((_C, _DP), jnp.float32),
            pltpu.VMEM((_C, _DP), jnp.float32),
            pltpu.VMEM((_C, _DP), jnp.float32),
            pltpu.SemaphoreType.DMA,
            pltpu.SemaphoreType.DMA,
            pltpu.SemaphoreType.DMA,
            pltpu.SemaphoreType.DMA,
            pltpu.SemaphoreType.DMA,
            pltpu.SemaphoreType.DMA,
        ],
    )
    def gather_sum(pidx_hbm, sidx_hbm, ptab_hbm, stab_hbm, out_hbm,
                   idxp_v, idxs_v, bufp0, bufp1, bufs0, bufs1, bufo0, bufo1,
                   semp0, semp1, sems0, sems1, semw0, semw1):
        bufp = (bufp0, bufp1)
        bufs = (bufs0, bufs1)
        bufo = (bufo0, bufo1)
        semp = (semp0, semp1)
        sems = (sems0, sems1)
        semw = (semw0, semw1)

        wid = lax.axis_index("s") * _NC + lax.axis_index("c")
        wbase = wid * _PER_W
        pltpu.sync_copy(pidx_hbm.at[pl.ds(wbase, _PER_W)], idxp_v)
        pltpu.sync_copy(sidx_hbm.at[pl.ds(wbase, _PER_W)], idxs_v)

        def gather_pair(g, b):
            pltpu.async_copy(
                ptab_hbm.at[idxp_v.at[pl.ds(g * _C, _C)]], bufp[b], semp[b])
            pltpu.async_copy(
                stab_hbm.at[idxs_v.at[pl.ds(g * _C, _C)]], bufs[b], sems[b])

        def wait_gather_pair(g, b):
            pltpu.make_async_copy(
                ptab_hbm.at[idxp_v.at[pl.ds(g * _C, _C)]], bufp[b], semp[b]).wait()
            pltpu.make_async_copy(
                stab_hbm.at[idxs_v.at[pl.ds(g * _C, _C)]], bufs[b], sems[b]).wait()

        gather_pair(0, 0)
        gather_pair(1, 1)

        def add_chunk(b):
            himask = jnp.uint32(0xFFFF0000).astype(jnp.int32)
            rnd = jnp.int32(0x8000)

            def row(c, carry):
                for j in range(_DP // _LANES):
                    o = j * _LANES
                    wp = plsc.bitcast(bufp[b][c, pl.ds(o, _LANES)], jnp.int32)
                    ws = plsc.bitcast(bufs[b][c, pl.ds(o, _LANES)], jnp.int32)
                    ap = plsc.bitcast(lax.shift_left(wp, 16), jnp.float32)
                    as_ = plsc.bitcast(lax.shift_left(ws, 16), jnp.float32)
                    bp = plsc.bitcast(lax.bitwise_and(wp, himask), jnp.float32)
                    bs = plsc.bitcast(lax.bitwise_and(ws, himask), jnp.float32)
                    abits = plsc.bitcast(ap + as_, jnp.int32)
                    bbits = plsc.bitcast(bp + bs, jnp.int32)
                    lo = lax.shift_right_logical(abits + rnd, 16)
                    hi = lax.bitwise_and(bbits + rnd, himask)
                    bufo[b][c, pl.ds(o, _LANES)] = plsc.bitcast(
                        lax.bitwise_or(lo, hi), jnp.float32)
                return carry
            lax.fori_loop(0, _C, row, 0)

        def outer(g2, carry):
            for b in (0, 1):
                g = g2 * 2 + b
                base = wbase + g * _C
                wait_gather_pair(g, b)
                @pl.when(g >= 2)
                def _():
                    pltpu.make_async_copy(
                        bufo[b], out_hbm.at[pl.ds(base, _C)], semw[b]).wait()
                add_chunk(b)
                pltpu.async_copy(bufo[b], out_hbm.at[pl.ds(base, _C)], semw[b])
                @pl.when(g + 2 < _G)
                def _():
                    gather_pair(g + 2, b)
            return carry

        lax.fori_loop(0, _G // 2, outer, 0)

        for b in (0, 1):
            pltpu.make_async_copy(
                bufo[b], out_hbm.at[pl.ds(wbase, _C)], semw[b]).wait()

    return gather_sum


_gather_sum = _make_gather_sum()


def _pack_body(p_ref, s_ref, po_ref, so_ref):
    himask = jnp.uint32(0xFFFF0000).astype(jnp.int32)

    def rne(x):
        xi = lax.bitcast_convert_type(x, jnp.int32)
        return xi + jnp.int32(0x7FFF) + lax.bitwise_and(
            lax.shift_right_logical(xi, 16), jnp.int32(1))

    def pack(x):
        lo = lax.shift_right_logical(rne(x[:, :_DP]), 16)
        hi = lax.bitwise_and(rne(x[:, _DP:]), himask)
        return lax.bitcast_convert_type(lax.bitwise_or(lo, hi), jnp.float32)

    po_ref[...] = pack(p_ref[...])
    so_ref[...] = pack(s_ref[...])


_VT = 200


def _pack_tables(ptab, stab):
    return pl.pallas_call(
        _pack_body,
        grid=(_V // _VT,),
        in_specs=[
            pl.BlockSpec((_VT, _D), lambda i: (i, 0)),
            pl.BlockSpec((_VT, _D), lambda i: (i, 0)),
        ],
        out_specs=[
            pl.BlockSpec((_VT, _DP), lambda i: (i, 0)),
            pl.BlockSpec((_VT, _DP), lambda i: (i, 0)),
        ],
        out_shape=[
            jax.ShapeDtypeStruct((_V, _DP), jnp.float32),
            jax.ShapeDtypeStruct((_V, _DP), jnp.float32),
        ],
    )(ptab, stab)


def _ln_body(e_ref, x_ref, y_ref, vecs_ref, o_ref):
    himask = jnp.uint32(0xFFFF0000).astype(jnp.int32)
    w = lax.bitcast_convert_type(e_ref[...], jnp.int32)
    elo = lax.bitcast_convert_type(lax.shift_left(w, 16), jnp.float32)
    ehi = lax.bitcast_convert_type(lax.bitwise_and(w, himask), jnp.float32)
    x = x_ref[...]
    y = y_ref[...]
    v = vecs_ref[...]
    elo = elo + x * v[0:1] + y * v[2:3] + v[4:5]
    ehi = ehi + x * v[1:2] + y * v[3:4] + v[5:6]
    mean = (jnp.sum(elo, axis=1, keepdims=True)
            + jnp.sum(ehi, axis=1, keepdims=True)) * (1.0 / _D)
    dlo = elo - mean
    dhi = ehi - mean
    var = (jnp.sum(dlo * dlo, axis=1, keepdims=True)
           + jnp.sum(dhi * dhi, axis=1, keepdims=True)) * (1.0 / _D)
    inv = lax.rsqrt(var + 1e-5)
    o_ref[:, :_DP] = dlo * inv * v[6:7] + v[8:9]
    o_ref[:, _DP:] = dhi * inv * v[7:8] + v[9:10]


_T = 128


def _ln_call(esum_pk, x2, y2, vecs):
    return pl.pallas_call(
        _ln_body,
        grid=(_N // _T,),
        in_specs=[
            pl.BlockSpec((_T, _DP), lambda i: (i, 0)),
            pl.BlockSpec((_T, 1), lambda i: (i, 0)),
            pl.BlockSpec((_T, 1), lambda i: (i, 0)),
            pl.BlockSpec((10, _DP), lambda i: (0, 0)),
        ],
        out_specs=pl.BlockSpec((_T, _D), lambda i: (i, 0)),
        out_shape=jax.ShapeDtypeStruct((_N, _D), jnp.float32),
    )(esum_pk, x2, y2, vecs)


def kernel(primary, ss, x, y, primary_table, ss_table, W_cord, b_cord, gamma, beta):
    pidx = primary.reshape(_N).astype(jnp.int32)
    sidx = ss.reshape(_N).astype(jnp.int32)
    ptab_pk, stab_pk = _pack_tables(primary_table, ss_table)
    esum_pk = _gather_sum(pidx, sidx, ptab_pk, stab_pk)
    wy = W_cord[1] + W_cord[2]
    vecs = jnp.stack([
        W_cord[0, :_DP], W_cord[0, _DP:],
        wy[:_DP], wy[_DP:],
        b_cord[:_DP], b_cord[_DP:],
        gamma[:_DP], gamma[_DP:],
        beta[:_DP], beta[_DP:],
    ])
    out = _ln_call(esum_pk, x.reshape(_N, 1), y.reshape(_N, 1), vecs)
    return out.reshape(_B, _L, _D)


# speedup vs baseline: 3.5099x; 1.2505x over previous
import functools

import jax
import jax.numpy as jnp
from jax import lax
from jax.experimental import pallas as pl
from jax.experimental.pallas import tpu as pltpu
from jax.experimental.pallas import tpu_sc as plsc

_B, _L, _V, _D = 4, 4096, 1000, 2048
_N = _B * _L
_NC, _NS = 2, 16
_NW = _NC * _NS
_PER_W = _N // _NW
_C = 16
_G = _PER_W // _C
_LANES = 16
_DP = _D // 2


def _make_gather_sum():
    mesh = plsc.VectorSubcoreMesh(
        core_axis_name="c", subcore_axis_name="s",
        num_cores=_NC, num_subcores=_NS)

    @functools.partial(
        pl.kernel,
        out_type=jax.ShapeDtypeStruct((_N, _DP), jnp.float32),
        mesh=mesh,
        compiler_params=pltpu.CompilerParams(needs_layout_passes=False),
        scratch_types=[
            pltpu.VMEM((_PER_W,), jnp.int32),
            pltpu.VMEM((_PER_W,), jnp.int32),
            pltpu.VMEM((_C, _DP), jnp.float32),
            pltpu.VMEM((_C, _DP), jnp.float32),
            pltpu.VMEM((_C, _DP), jnp.float32),
            pltpu.VMEM((_C, _DP), jnp.float32),
            pltpu.VMEM((_C, _DP), jnp.float32),
            pltpu.VMEM((_C, _DP), jnp.float32),
            pltpu.SemaphoreType.DMA,
            pltpu.SemaphoreType.DMA,
            pltpu.SemaphoreType.DMA,
            pltpu.SemaphoreType.DMA,
            pltpu.SemaphoreType.DMA,
            pltpu.SemaphoreType.DMA,
        ],
    )
    def gather_sum(pidx_hbm, sidx_hbm, ptab_hbm, stab_hbm, out_hbm,
                   idxp_v, idxs_v, bufp0, bufp1, bufs0, bufs1, bufo0, bufo1,
                   semp0, semp1, sems0, sems1, semw0, semw1):
        bufp = (bufp0, bufp1)
        bufs = (bufs0, bufs1)
        bufo = (bufo0, bufo1)
        semp = (semp0, semp1)
        sems = (sems0, sems1)
        semw = (semw0, semw1)

        wid = lax.axis_index("s") * _NC + lax.axis_index("c")
        wbase = wid * _PER_W
        pltpu.sync_copy(pidx_hbm.at[pl.ds(wbase, _PER_W)], idxp_v)
        pltpu.sync_copy(sidx_hbm.at[pl.ds(wbase, _PER_W)], idxs_v)

        def gather_pair(g, b):
            pltpu.async_copy(
                ptab_hbm.at[idxp_v.at[pl.ds(g * _C, _C)]], bufp[b], semp[b])
            pltpu.async_copy(
                stab_hbm.at[idxs_v.at[pl.ds(g * _C, _C)]], bufs[b], sems[b])

        def wait_gather_pair(g, b):
            pltpu.make_async_copy(
                ptab_hbm.at[idxp_v.at[pl.ds(g * _C, _C)]], bufp[b], semp[b]).wait()
            pltpu.make_async_copy(
                stab_hbm.at[idxs_v.at[pl.ds(g * _C, _C)]], bufs[b], sems[b]).wait()

        gather_pair(0, 0)
        gather_pair(1, 1)

        def add_chunk(b):
            himask = jnp.uint32(0xFFFF0000).astype(jnp.int32)
            rnd = jnp.int32(0x8000)

            def row(c, carry):
                for j in range(_DP // _LANES):
                    o = j * _LANES
                    wp = plsc.bitcast(bufp[b][c, pl.ds(o, _LANES)], jnp.int32)
                    ws = plsc.bitcast(bufs[b][c, pl.ds(o, _LANES)], jnp.int32)
                    ap = plsc.bitcast(lax.shift_left(wp, 16), jnp.float32)
                    as_ = plsc.bitcast(lax.shift_left(ws, 16), jnp.float32)
                    bp = plsc.bitcast(lax.bitwise_and(wp, himask), jnp.float32)
                    bs = plsc.bitcast(lax.bitwise_and(ws, himask), jnp.float32)
                    abits = plsc.bitcast(ap + as_, jnp.int32)
                    bbits = plsc.bitcast(bp + bs, jnp.int32)
                    lo = lax.shift_right_logical(abits + rnd, 16)
                    hi = lax.bitwise_and(bbits + rnd, himask)
                    bufo[b][c, pl.ds(o, _LANES)] = plsc.bitcast(
                        lax.bitwise_or(lo, hi), jnp.float32)
                return carry
            lax.fori_loop(0, _C, row, 0)

        def outer(g2, carry):
            for b in (0, 1):
                g = g2 * 2 + b
                base = wbase + g * _C
                wait_gather_pair(g, b)
                @pl.when(g >= 2)
                def _():
                    pltpu.make_async_copy(
                        bufo[b], out_hbm.at[pl.ds(base, _C)], semw[b]).wait()
                add_chunk(b)
                pltpu.async_copy(bufo[b], out_hbm.at[pl.ds(base, _C)], semw[b])
                @pl.when(g + 2 < _G)
                def _():
                    gather_pair(g + 2, b)
            return carry

        lax.fori_loop(0, _G // 2, outer, 0)

        for b in (0, 1):
            pltpu.make_async_copy(
                bufo[b], out_hbm.at[pl.ds(wbase, _C)], semw[b]).wait()

    return gather_sum


_gather_sum = _make_gather_sum()


def _pack_body(p_ref, s_ref, po_ref, so_ref):
    himask = jnp.uint32(0xFFFF0000).astype(jnp.int32)

    def rne(x):
        xi = lax.bitcast_convert_type(x, jnp.int32)
        return xi + jnp.int32(0x7FFF) + lax.bitwise_and(
            lax.shift_right_logical(xi, 16), jnp.int32(1))

    def pack(x):
        lo = lax.shift_right_logical(rne(x[:, :_DP]), 16)
        hi = lax.bitwise_and(rne(x[:, _DP:]), himask)
        return lax.bitcast_convert_type(lax.bitwise_or(lo, hi), jnp.float32)

    po_ref[...] = pack(p_ref[...])
    so_ref[...] = pack(s_ref[...])


_VT = 200


def _pack_tables(ptab, stab):
    return pl.pallas_call(
        _pack_body,
        grid=(_V // _VT,),
        in_specs=[
            pl.BlockSpec((_VT, _D), lambda i: (i, 0)),
            pl.BlockSpec((_VT, _D), lambda i: (i, 0)),
        ],
        out_specs=[
            pl.BlockSpec((_VT, _DP), lambda i: (i, 0)),
            pl.BlockSpec((_VT, _DP), lambda i: (i, 0)),
        ],
        out_shape=[
            jax.ShapeDtypeStruct((_V, _DP), jnp.float32),
            jax.ShapeDtypeStruct((_V, _DP), jnp.float32),
        ],
    )(ptab, stab)


def _ln_body(e_ref, x_ref, y_ref, vecs_ref, o_ref):
    himask = jnp.uint32(0xFFFF0000).astype(jnp.int32)
    w = lax.bitcast_convert_type(e_ref[...], jnp.int32)
    elo = lax.bitcast_convert_type(lax.shift_left(w, 16), jnp.float32)
    ehi = lax.bitcast_convert_type(lax.bitwise_and(w, himask), jnp.float32)
    x = x_ref[...]
    y = y_ref[...]
    v = vecs_ref[...]
    elo = elo + x * v[0:1] + y * v[2:3] + v[4:5]
    ehi = ehi + x * v[1:2] + y * v[3:4] + v[5:6]
    mean = (jnp.sum(elo, axis=1, keepdims=True)
            + jnp.sum(ehi, axis=1, keepdims=True)) * (1.0 / _D)
    dlo = elo - mean
    dhi = ehi - mean
    var = (jnp.sum(dlo * dlo, axis=1, keepdims=True)
           + jnp.sum(dhi * dhi, axis=1, keepdims=True)) * (1.0 / _D)
    inv = lax.rsqrt(var + 1e-5)
    o_ref[:, :_DP] = dlo * inv * v[6:7] + v[8:9]
    o_ref[:, _DP:] = dhi * inv * v[7:8] + v[9:10]


_T = 512


def _ln_call(esum_pk, x2, y2, vecs):
    return pl.pallas_call(
        _ln_body,
        grid=(_N // _T,),
        in_specs=[
            pl.BlockSpec((_T, _DP), lambda i: (i, 0)),
            pl.BlockSpec((_T, 1), lambda i: (i, 0)),
            pl.BlockSpec((_T, 1), lambda i: (i, 0)),
            pl.BlockSpec((10, _DP), lambda i: (0, 0)),
        ],
        out_specs=pl.BlockSpec((_T, _D), lambda i: (i, 0)),
        out_shape=jax.ShapeDtypeStruct((_N, _D), jnp.float32),
    )(esum_pk, x2, y2, vecs)


def kernel(primary, ss, x, y, primary_table, ss_table, W_cord, b_cord, gamma, beta):
    pidx = primary.reshape(_N).astype(jnp.int32)
    sidx = ss.reshape(_N).astype(jnp.int32)
    ptab_pk, stab_pk = _pack_tables(primary_table, ss_table)
    esum_pk = _gather_sum(pidx, sidx, ptab_pk, stab_pk)
    wy = W_cord[1] + W_cord[2]
    vecs = jnp.stack([
        W_cord[0, :_DP], W_cord[0, _DP:],
        wy[:_DP], wy[_DP:],
        b_cord[:_DP], b_cord[_DP:],
        gamma[:_DP], gamma[_DP:],
        beta[:_DP], beta[_DP:],
    ])
    out = _ln_call(esum_pk, x.reshape(_N, 1), y.reshape(_N, 1), vecs)
    return out.reshape(_B, _L, _D)
